# baseline (speedup 1.0000x reference)
import jax
import jax.numpy as jnp
from jax import lax
from jax.experimental import pallas as pl
from jax.experimental.pallas import tpu as pltpu

L = 200
D = 64
TW = 128
LT = L * TW
V2 = 1000
NQ = 4
LQ = L // NQ


def _select_sum(ivexp_ref, tiles):
    lane = lax.broadcasted_iota(jnp.int32, (1, LT), 1) % TW
    m = (ivexp_ref[...] == lane).astype(jnp.float32)
    sel = tiles[...] * m
    return jnp.sum(sel, axis=1, keepdims=True)


def _fire(idx_smem, et_hbm, tiles, sems):
    def go(i, _):
        for q in range(NQ):
            k = i * NQ + q
            s = pl.multiple_of((idx_smem[0, k] // TW) * TW, TW)
            d = pl.multiple_of(k * TW, TW)
            pltpu.make_async_copy(et_hbm.at[:, pl.ds(s, TW)],
                                  tiles.at[:, pl.ds(d, TW)],
                                  sems[q]).start()
        return _
    lax.fori_loop(0, LQ, go, None)


def _drain(et_hbm, tiles, sems):
    def go(i, _):
        for q in range(NQ):
            d = pl.multiple_of((i * NQ + q) * TW, TW)
            pltpu.make_async_copy(et_hbm.at[:, pl.ds(0, TW)],
                                  tiles.at[:, pl.ds(d, TW)],
                                  sems[q]).wait()
        return _
    lax.fori_loop(0, LQ, go, None)


def _body(i0_ref, i1_ref, med_ref, ie0_ref, ie1_ref, e0t_hbm, e1t_hbm,
          e2t_ref, wt_ref, b_ref, ddi_ref, res_ref, bn_ref,
          tiles0, tiles1, *sems):
    _fire(i0_ref, e0t_hbm, tiles0, sems[:NQ])
    _fire(i1_ref, e1t_hbm, tiles1, sems[NQ:])

    iota = lax.broadcasted_iota(jnp.int32, (1, V2), 1)
    eq = (med_ref[...] == iota).astype(jnp.float32)
    counts = jnp.sum(eq, axis=0, keepdims=True)
    i3 = lax.dot_general(
        e2t_ref[...], counts, (((1,), (1,)), ((), ())),
        preferred_element_type=jnp.float32,
        precision=lax.Precision.HIGHEST)

    _drain(e0t_hbm, tiles0, sems[:NQ])
    i1 = _select_sum(ie0_ref, tiles0)
    _drain(e1t_hbm, tiles1, sems[NQ:])
    i2 = _select_sum(ie1_ref, tiles1)

    x = jnp.concatenate([i1, i2, i3], axis=0)
    x = jnp.maximum(x, 0.0)
    r = lax.dot_general(
        x, wt_ref[...], (((0,), (0,)), ((), ())),
        preferred_element_type=jnp.float32,
        precision=lax.Precision.HIGHEST) + b_ref[...]
    res_ref[...] = r
    p = jax.nn.sigmoid(r)
    v = jnp.dot(p, ddi_ref[...],
                preferred_element_type=jnp.float32,
                precision=lax.Precision.HIGHEST)
    bn_ref[...] = (0.0005 * jnp.sum(v * p))[None, None]


def kernel(diag_idx, proc_idx, med_idx, emb0, emb1, emb2, W, b, ddi_adj):
    i0 = diag_idx.astype(jnp.int32).reshape(1, L)
    i1 = proc_idx.astype(jnp.int32).reshape(1, L)
    med_col = med_idx.astype(jnp.int32).reshape(L, 1)
    ie0 = jnp.repeat(i0[0] % TW, TW).reshape(1, LT)
    ie1 = jnp.repeat(i1[0] % TW, TW).reshape(1, LT)
    res, bn = pl.pallas_call(
        _body,
        in_specs=[
            pl.BlockSpec(memory_space=pltpu.MemorySpace.SMEM),
            pl.BlockSpec(memory_space=pltpu.MemorySpace.SMEM),
            pl.BlockSpec(memory_space=pltpu.MemorySpace.VMEM),
            pl.BlockSpec(memory_space=pltpu.MemorySpace.VMEM),
            pl.BlockSpec(memory_space=pltpu.MemorySpace.VMEM),
            pl.BlockSpec(memory_space=pltpu.MemorySpace.HBM),
            pl.BlockSpec(memory_space=pltpu.MemorySpace.HBM),
            pl.BlockSpec(memory_space=pltpu.MemorySpace.VMEM),
            pl.BlockSpec(memory_space=pltpu.MemorySpace.VMEM),
            pl.BlockSpec(memory_space=pltpu.MemorySpace.VMEM),
            pl.BlockSpec(memory_space=pltpu.MemorySpace.VMEM),
        ],
        out_shape=(jax.ShapeDtypeStruct((1, V2), jnp.float32),
                   jax.ShapeDtypeStruct((1, 1), jnp.float32)),
        scratch_shapes=([pltpu.VMEM((D, LT), jnp.float32),
                         pltpu.VMEM((D, LT), jnp.float32)]
                        + [pltpu.SemaphoreType.DMA] * (2 * NQ)),
    )(i0, i1, med_col, ie0, ie1, emb0.T, emb1.T, emb2.T, W.T,
      b.reshape(1, V2), ddi_adj)
    return res, bn[0, 0]


# speedup vs baseline: 1.0725x; 1.0725x over previous
import jax
import jax.numpy as jnp
from jax import lax
from jax.experimental import pallas as pl
from jax.experimental.pallas import tpu as pltpu

L = 200
D = 64
TW = 128
LT = L * TW
V2 = 1000
NQ = 4
LQ = L // NQ


def _select_sum(ivexp_ref, tiles):
    lane = lax.broadcasted_iota(jnp.int32, (1, LT), 1) % TW
    m = (ivexp_ref[...] == lane).astype(jnp.float32)
    sel = tiles[...] * m
    return jnp.sum(sel, axis=1, keepdims=True)


def _fire(idx_smem, et_hbm, tiles, sems):
    def go(i, _):
        for q in range(NQ):
            k = i * NQ + q
            s = pl.multiple_of((idx_smem[0, k] // TW) * TW, TW)
            d = pl.multiple_of(k * TW, TW)
            pltpu.make_async_copy(et_hbm.at[:, pl.ds(s, TW)],
                                  tiles.at[:, pl.ds(d, TW)],
                                  sems[q]).start()
        return _
    lax.fori_loop(0, LQ, go, None)


def _drain(et_hbm, tiles, sems):
    def go(i, _):
        for q in range(NQ):
            d = pl.multiple_of((i * NQ + q) * TW, TW)
            pltpu.make_async_copy(et_hbm.at[:, pl.ds(0, TW)],
                                  tiles.at[:, pl.ds(d, TW)],
                                  sems[q]).wait()
        return _
    lax.fori_loop(0, LQ, go, None)


def _body(i0_ref, i1_ref, iv_ref, ie_ref, e0t_hbm, e1t_hbm,
          e2t_ref, wt_ref, b_ref, ddi_ref, res_ref, bn_ref,
          tiles0, tiles1, *sems):
    _fire(i0_ref, e0t_hbm, tiles0, sems[:NQ])
    _fire(i1_ref, e1t_hbm, tiles1, sems[NQ:])

    iota = lax.broadcasted_iota(jnp.int32, (1, V2), 1)
    eq = (iv_ref[:, 2:3] == iota).astype(jnp.float32)
    counts = jnp.sum(eq, axis=0, keepdims=True)
    i3 = lax.dot_general(
        e2t_ref[...], counts, (((1,), (1,)), ((), ())),
        preferred_element_type=jnp.float32,
        precision=lax.Precision.HIGHEST)

    _drain(e0t_hbm, tiles0, sems[:NQ])
    i1 = _select_sum(ie_ref.at[:, :LT], tiles0)
    _drain(e1t_hbm, tiles1, sems[NQ:])
    i2 = _select_sum(ie_ref.at[:, LT:], tiles1)

    x = jnp.concatenate([i1, i2, i3], axis=0)
    x = jnp.maximum(x, 0.0)
    r = lax.dot_general(
        x, wt_ref[...], (((0,), (0,)), ((), ())),
        preferred_element_type=jnp.float32,
        precision=lax.Precision.HIGHEST) + b_ref[...]
    res_ref[...] = r
    p = jax.nn.sigmoid(r)
    v = jnp.dot(p, ddi_ref[...],
                preferred_element_type=jnp.float32,
                precision=lax.Precision.HIGHEST)
    bn_ref[...] = (0.0005 * jnp.sum(v * p))[None, None]


def kernel(diag_idx, proc_idx, med_idx, emb0, emb1, emb2, W, b, ddi_adj):
    i0 = diag_idx.astype(jnp.int32).reshape(1, L)
    i1 = proc_idx.astype(jnp.int32).reshape(1, L)
    iv = jnp.stack([diag_idx, proc_idx, med_idx], axis=1).astype(jnp.int32)
    ie = jnp.repeat(
        jnp.concatenate([i0[0], i1[0]]) % TW, TW).reshape(1, 2 * LT)
    res, bn = pl.pallas_call(
        _body,
        in_specs=[
            pl.BlockSpec(memory_space=pltpu.MemorySpace.SMEM),
            pl.BlockSpec(memory_space=pltpu.MemorySpace.SMEM),
            pl.BlockSpec(memory_space=pltpu.MemorySpace.VMEM),
            pl.BlockSpec(memory_space=pltpu.MemorySpace.VMEM),
            pl.BlockSpec(memory_space=pltpu.MemorySpace.HBM),
            pl.BlockSpec(memory_space=pltpu.MemorySpace.HBM),
            pl.BlockSpec(memory_space=pltpu.MemorySpace.VMEM),
            pl.BlockSpec(memory_space=pltpu.MemorySpace.VMEM),
            pl.BlockSpec(memory_space=pltpu.MemorySpace.VMEM),
            pl.BlockSpec(memory_space=pltpu.MemorySpace.VMEM),
        ],
        out_shape=(jax.ShapeDtypeStruct((1, V2), jnp.float32),
                   jax.ShapeDtypeStruct((1, 1), jnp.float32)),
        scratch_shapes=([pltpu.VMEM((D, LT), jnp.float32),
                         pltpu.VMEM((D, LT), jnp.float32)]
                        + [pltpu.SemaphoreType.DMA] * (2 * NQ)),
    )(i0, i1, iv, ie, emb0.T, emb1.T, emb2.T, W.T,
      b.reshape(1, V2), ddi_adj)
    return res, bn[0, 0]


# speedup vs baseline: 1.0743x; 1.0017x over previous
import jax
import jax.numpy as jnp
from jax import lax
from jax.experimental import pallas as pl
from jax.experimental.pallas import tpu as pltpu

L = 200
D = 64
TW = 128
LT = L * TW
V2 = 1000
NQ = 4
LQ = L // NQ


def _select_sum(ivexp_ref, tiles):
    lane = lax.broadcasted_iota(jnp.int32, (1, LT), 1) % TW
    m = ivexp_ref[...] == lane
    sel = jnp.where(m, tiles[...], 0.0)
    return jnp.sum(sel, axis=1, keepdims=True)


def _fire(idx_smem, et_hbm, tiles, sems):
    def go(i, _):
        for q in range(NQ):
            k = i * NQ + q
            s = pl.multiple_of((idx_smem[0, k] // TW) * TW, TW)
            d = pl.multiple_of(k * TW, TW)
            pltpu.make_async_copy(et_hbm.at[:, pl.ds(s, TW)],
                                  tiles.at[:, pl.ds(d, TW)],
                                  sems[q]).start()
        return _
    lax.fori_loop(0, LQ, go, None)


def _drain(et_hbm, tiles, sems):
    def go(i, _):
        for q in range(NQ):
            d = pl.multiple_of((i * NQ + q) * TW, TW)
            pltpu.make_async_copy(et_hbm.at[:, pl.ds(0, TW)],
                                  tiles.at[:, pl.ds(d, TW)],
                                  sems[q]).wait()
        return _
    lax.fori_loop(0, LQ, go, None)


def _body(i0_ref, i1_ref, iv_ref, ie_ref, e0t_hbm, e1t_hbm,
          e2t_ref, wt_ref, b_ref, ddi_ref, res_ref, bn_ref,
          tiles0, tiles1, *sems):
    _fire(i0_ref, e0t_hbm, tiles0, sems[:NQ])
    _fire(i1_ref, e1t_hbm, tiles1, sems[NQ:])

    iota = lax.broadcasted_iota(jnp.int32, (1, V2), 1)
    eq = (iv_ref[:, 2:3] == iota).astype(jnp.float32)
    counts = jnp.sum(eq, axis=0, keepdims=True)
    i3 = lax.dot_general(
        e2t_ref[...], counts, (((1,), (1,)), ((), ())),
        preferred_element_type=jnp.float32,
        precision=lax.Precision.HIGHEST)

    _drain(e0t_hbm, tiles0, sems[:NQ])
    i1 = _select_sum(ie_ref.at[:, :LT], tiles0)
    _drain(e1t_hbm, tiles1, sems[NQ:])
    i2 = _select_sum(ie_ref.at[:, LT:], tiles1)

    x = jnp.concatenate([i1, i2, i3], axis=0)
    x = jnp.maximum(x, 0.0)
    r = lax.dot_general(
        x, wt_ref[...], (((0,), (0,)), ((), ())),
        preferred_element_type=jnp.float32,
        precision=lax.Precision.HIGHEST) + b_ref[...]
    res_ref[...] = r
    p = jax.nn.sigmoid(r)
    v = jnp.dot(p, ddi_ref[...],
                preferred_element_type=jnp.float32,
                precision=lax.Precision.HIGHEST)
    bn_ref[...] = (0.0005 * jnp.sum(v * p))[None, None]


def kernel(diag_idx, proc_idx, med_idx, emb0, emb1, emb2, W, b, ddi_adj):
    i0 = diag_idx.astype(jnp.int32).reshape(1, L)
    i1 = proc_idx.astype(jnp.int32).reshape(1, L)
    iv = jnp.stack([diag_idx, proc_idx, med_idx], axis=1).astype(jnp.int32)
    ie = jnp.repeat(
        jnp.concatenate([i0[0], i1[0]]) % TW, TW).reshape(1, 2 * LT)
    res, bn = pl.pallas_call(
        _body,
        in_specs=[
            pl.BlockSpec(memory_space=pltpu.MemorySpace.SMEM),
            pl.BlockSpec(memory_space=pltpu.MemorySpace.SMEM),
            pl.BlockSpec(memory_space=pltpu.MemorySpace.VMEM),
            pl.BlockSpec(memory_space=pltpu.MemorySpace.VMEM),
            pl.BlockSpec(memory_space=pltpu.MemorySpace.HBM),
            pl.BlockSpec(memory_space=pltpu.MemorySpace.HBM),
            pl.BlockSpec(memory_space=pltpu.MemorySpace.VMEM),
            pl.BlockSpec(memory_space=pltpu.MemorySpace.VMEM),
            pl.BlockSpec(memory_space=pltpu.MemorySpace.VMEM),
            pl.BlockSpec(memory_space=pltpu.MemorySpace.VMEM),
        ],
        out_shape=(jax.ShapeDtypeStruct((1, V2), jnp.float32),
                   jax.ShapeDtypeStruct((1, 1), jnp.float32)),
        scratch_shapes=([pltpu.VMEM((D, LT), jnp.float32),
                         pltpu.VMEM((D, LT), jnp.float32)]
                        + [pltpu.SemaphoreType.DMA] * (2 * NQ)),
    )(i0, i1, iv, ie, emb0.T, emb1.T, emb2.T, W.T,
      b.reshape(1, V2), ddi_adj)
    return res, bn[0, 0]
